# Initial kernel scaffold; baseline (speedup 1.0000x reference)
import jax
import jax.numpy as jnp
from jax.experimental import pallas as pl


def kernel(x, edge_index, W0, b0, W1, b1, W2, b2):
    raise NotImplementedError("write your pallas kernel here")


# speedup vs baseline: 13.1275x; 13.1275x over previous
import functools

import jax
import jax.numpy as jnp
from jax import lax
from jax.experimental import pallas as pl
from jax.experimental.pallas import tpu as pltpu
from jax.experimental.pallas import tpu_sc as plsc

NC = 2
NS = 16
CH = 128


def _sc_mesh():
    return plsc.VectorSubcoreMesh(core_axis_name="c", subcore_axis_name="s")


def _row_split(n):
    big = -(-n // NS)
    big = -(-big // 8) * 8
    last = n - (NS - 1) * big
    assert last > 0 and last % 8 == 0 and big % 8 == 0
    return big, last


def _tile_rows_copy(s, big, last, copy_big, copy_last):
    @pl.when(s < NS - 1)
    def _():
        copy_big()

    @pl.when(s == NS - 1)
    def _():
        copy_last()


def _make_deg_kernel(E, n):
    EC = E // NC
    ET = EC // NS
    assert EC * NC == E and ET * NS == EC
    n_full = ET // CH
    tail = ET - n_full * CH
    big, last = _row_split(n)

    @functools.partial(
        pl.kernel,
        out_type=jax.ShapeDtypeStruct((NC, n, 128), jnp.float32),
        mesh=_sc_mesh(),
        scratch_types=[
            pltpu.VMEM_SHARED((n, 128), jnp.float32),
            pltpu.VMEM((CH, 128), jnp.float32),
            pltpu.VMEM((CH,), jnp.int32),
            pltpu.VMEM((max(tail, 8), 128), jnp.float32),
            pltpu.VMEM((max(tail, 8),), jnp.int32),
        ],
    )
    def deg_kernel(dst_hbm, ones_hbm, zeros_hbm, out_hbm,
                   deg_sh, ones_v, dst_v, ones_t, dst_t):
        c = lax.axis_index("c")
        s = lax.axis_index("s")
        base = c * EC + s * ET
        row0 = s * big
        _tile_rows_copy(
            s, big, last,
            lambda: pltpu.sync_copy(zeros_hbm.at[pl.ds(0, big)],
                                    deg_sh.at[pl.ds(row0, big)]),
            lambda: pltpu.sync_copy(zeros_hbm.at[pl.ds(0, last)],
                                    deg_sh.at[pl.ds(row0, last)]))
        pltpu.sync_copy(ones_hbm, ones_v)
        if tail:
            pltpu.sync_copy(ones_hbm.at[pl.ds(0, tail)], ones_t)
        plsc.subcore_barrier()

        def body(i, carry):
            off = base + i * CH
            pltpu.sync_copy(dst_hbm.at[pl.ds(off, CH)], dst_v)
            pltpu.sync_copy(ones_v, deg_sh.at[dst_v], add=True)
            return carry

        lax.fori_loop(0, n_full, body, 0)
        if tail:
            off = base + n_full * CH
            pltpu.sync_copy(dst_hbm.at[pl.ds(off, tail)], dst_t)
            pltpu.sync_copy(ones_t, deg_sh.at[dst_t], add=True)
        plsc.subcore_barrier()
        _tile_rows_copy(
            s, big, last,
            lambda: pltpu.sync_copy(deg_sh.at[pl.ds(row0, big)],
                                    out_hbm.at[c, pl.ds(row0, big)]),
            lambda: pltpu.sync_copy(deg_sh.at[pl.ds(row0, last)],
                                    out_hbm.at[c, pl.ds(row0, last)]))

    return deg_kernel


def _make_agg_kernel(E, n, F):
    EC = E // NC
    ET = EC // NS
    assert EC * NC == E and ET * NS == EC
    n_full = ET // CH
    tail = ET - n_full * CH
    big, last = _row_split(n)

    @functools.partial(
        pl.kernel,
        out_type=jax.ShapeDtypeStruct((NC, n, F), jnp.float32),
        mesh=_sc_mesh(),
        scratch_types=[
            pltpu.VMEM_SHARED((n, F), jnp.float32),
            pltpu.VMEM((CH,), jnp.int32),
            pltpu.VMEM((CH,), jnp.int32),
            pltpu.VMEM((CH, F), jnp.float32),
            pltpu.VMEM((max(tail, 8),), jnp.int32),
            pltpu.VMEM((max(tail, 8),), jnp.int32),
            pltpu.VMEM((max(tail, 8), F), jnp.float32),
            pltpu.SemaphoreType.DMA,
        ],
    )
    def agg_kernel(hs_hbm, src_hbm, dst_hbm, zeros_hbm, out_hbm,
                   agg_sh, src_v, dst_v, rows_v, src_t, dst_t, rows_t, sem):
        c = lax.axis_index("c")
        s = lax.axis_index("s")
        base = c * EC + s * ET
        row0 = s * big
        _tile_rows_copy(
            s, big, last,
            lambda: pltpu.sync_copy(zeros_hbm.at[pl.ds(0, big)],
                                    agg_sh.at[pl.ds(row0, big)]),
            lambda: pltpu.sync_copy(zeros_hbm.at[pl.ds(0, last)],
                                    agg_sh.at[pl.ds(row0, last)]))
        plsc.subcore_barrier()

        def body(i, carry):
            off = base + i * CH
            pltpu.sync_copy(src_hbm.at[pl.ds(off, CH)], src_v)
            pltpu.sync_copy(dst_hbm.at[pl.ds(off, CH)], dst_v)
            pltpu.async_copy(hs_hbm.at[src_v], rows_v, sem).wait()
            pltpu.sync_copy(rows_v, agg_sh.at[dst_v], add=True)
            return carry

        lax.fori_loop(0, n_full, body, 0)
        if tail:
            off = base + n_full * CH
            pltpu.sync_copy(src_hbm.at[pl.ds(off, tail)], src_t)
            pltpu.sync_copy(dst_hbm.at[pl.ds(off, tail)], dst_t)
            pltpu.async_copy(hs_hbm.at[src_t], rows_t, sem).wait()
            pltpu.sync_copy(rows_t, agg_sh.at[dst_t], add=True)
        plsc.subcore_barrier()
        _tile_rows_copy(
            s, big, last,
            lambda: pltpu.sync_copy(agg_sh.at[pl.ds(row0, big)],
                                    out_hbm.at[c, pl.ds(row0, big)]),
            lambda: pltpu.sync_copy(agg_sh.at[pl.ds(row0, last)],
                                    out_hbm.at[c, pl.ds(row0, last)]))

    return agg_kernel


def _b0_body(x_ref, w_ref, degp_ref, dinvb_ref, hs_ref):
    deg = degp_ref[0] + degp_ref[1]
    deg0 = deg[:, 0:1] + 1.0
    dinv = lax.rsqrt(deg0)
    dinvb = jnp.broadcast_to(dinv, (dinv.shape[0], 128))
    dinvb_ref[...] = dinvb
    h = jnp.dot(x_ref[...], w_ref[...], preferred_element_type=jnp.float32)
    hs_ref[...] = h * dinvb


def _mid_body(aggp_ref, hs_ref, dinvb_ref, b_ref, w_ref, out_ref):
    t = (aggp_ref[0] + aggp_ref[1] + hs_ref[...]) * dinvb_ref[:, : hs_ref.shape[1]]
    t = t + b_ref[...]
    t = t * jax.nn.sigmoid(t)
    h = jnp.dot(t, w_ref[...], preferred_element_type=jnp.float32)
    out_ref[...] = h * dinvb_ref[:, : out_ref.shape[1]]


def _b3_body(aggp_ref, hs_ref, dinvb_ref, b_ref, out_ref):
    w = hs_ref.shape[1]
    z = (aggp_ref[0] + aggp_ref[1] + hs_ref[...]) * dinvb_ref[:, :w] + b_ref[...]
    m = jnp.max(z, axis=0, keepdims=True)
    lse = jnp.log(jnp.sum(jnp.exp(z - m), axis=0, keepdims=True)) + m
    out_ref[...] = (z - lse)[:, : out_ref.shape[1]]


def kernel(x, edge_index, W0, b0, W1, b1, W2, b2):
    n, f_in = x.shape
    E = edge_index.shape[1]
    C = W2.shape[1]
    f32 = jnp.float32
    src = edge_index[0].astype(jnp.int32)
    dst = edge_index[1].astype(jnp.int32)

    big, _ = _row_split(n)
    ones128 = jnp.ones((CH, 128), f32)
    z128 = jnp.zeros((big, 128), f32)

    degp = _make_deg_kernel(E, n)(dst, ones128, z128)

    dinvb, hs0 = pl.pallas_call(
        _b0_body,
        out_shape=[jax.ShapeDtypeStruct((n, 128), f32),
                   jax.ShapeDtypeStruct((n, 128), f32)],
    )(x, W0, degp)

    agg128 = _make_agg_kernel(E, n, 128)
    agg0 = agg128(hs0, src, dst, z128)

    hs1 = pl.pallas_call(
        _mid_body, out_shape=jax.ShapeDtypeStruct((n, 128), f32),
    )(agg0, hs0, dinvb, b0.reshape(1, 128), W1)

    agg1 = agg128(hs1, src, dst, z128)

    W2p = jnp.pad(W2, ((0, 0), (0, 128 - C)))
    b2p = jnp.pad(b2, (0, 128 - C)).reshape(1, 128)
    hs2 = pl.pallas_call(
        _mid_body, out_shape=jax.ShapeDtypeStruct((n, 128), f32),
    )(agg1, hs1, dinvb, b1.reshape(1, 128), W2p)

    agg2 = agg128(hs2, src, dst, z128)

    out = pl.pallas_call(
        _b3_body, out_shape=jax.ShapeDtypeStruct((n, C), f32),
    )(agg2, hs2, dinvb, b2p)
    return out


# speedup vs baseline: 20.7166x; 1.5781x over previous
import functools

import jax
import jax.numpy as jnp
from jax import lax
from jax.experimental import pallas as pl
from jax.experimental.pallas import tpu as pltpu
from jax.experimental.pallas import tpu_sc as plsc

NC = 2
NS = 16
CH = 64


def _sc_mesh():
    return plsc.VectorSubcoreMesh(core_axis_name="c", subcore_axis_name="s")


def _row_split(n):
    big = -(-n // NS)
    big = -(-big // 8) * 8
    last = n - (NS - 1) * big
    assert last > 0 and last % 8 == 0 and big % 8 == 0
    return big, last


def _tile_rows_copy(s, big, last, copy_big, copy_last):
    @pl.when(s < NS - 1)
    def _():
        copy_big()

    @pl.when(s == NS - 1)
    def _():
        copy_last()


def _make_deg_kernel(E, n):
    EC = E // NC
    ET = EC // NS
    assert EC * NC == E and ET * NS == EC
    n_full = ET // CH
    tail = ET - n_full * CH
    big, last = _row_split(n)

    K = 4
    n_body = n_full // K
    assert n_body * K == n_full

    @functools.partial(
        pl.kernel,
        out_type=jax.ShapeDtypeStruct((NC, n, 128), jnp.float32),
        mesh=_sc_mesh(),
        scratch_types=[
            pltpu.VMEM_SHARED((n, 128), jnp.float32),
            pltpu.VMEM((CH, 128), jnp.float32),
            [pltpu.VMEM((CH,), jnp.int32) for _ in range(K)],
            pltpu.VMEM((max(tail, 8), 128), jnp.float32),
            pltpu.VMEM((max(tail, 8),), jnp.int32),
            pltpu.SemaphoreType.DMA,
            pltpu.SemaphoreType.DMA,
        ],
    )
    def deg_kernel(dst_hbm, ones_hbm, zeros_hbm, out_hbm,
                   deg_sh, ones_v, dst_vs, ones_t, dst_t, sem_d, sem_s):
        c = lax.axis_index("c")
        s = lax.axis_index("s")
        base = c * EC + s * ET
        row0 = s * big
        _tile_rows_copy(
            s, big, last,
            lambda: pltpu.sync_copy(zeros_hbm.at[pl.ds(0, big)],
                                    deg_sh.at[pl.ds(row0, big)]),
            lambda: pltpu.sync_copy(zeros_hbm.at[pl.ds(0, last)],
                                    deg_sh.at[pl.ds(row0, last)]))
        pltpu.sync_copy(ones_hbm, ones_v)
        if tail:
            pltpu.sync_copy(ones_hbm.at[pl.ds(0, tail)], ones_t)
        plsc.subcore_barrier()

        @pl.loop(0, n_body)
        def _(i):
            off = base + i * (K * CH)
            ddescs = [
                pltpu.async_copy(dst_hbm.at[pl.ds(off + b * CH, CH)],
                                 dst_vs[b], sem_d)
                for b in range(K)
            ]
            sdescs = []
            for b in range(K):
                ddescs[b].wait()
                sdescs.append(
                    pltpu.async_copy(ones_v, deg_sh.at[dst_vs[b]],
                                     sem_s, add=True))
            for d in sdescs:
                d.wait()

        if tail:
            off = base + n_full * CH
            pltpu.sync_copy(dst_hbm.at[pl.ds(off, tail)], dst_t)
            pltpu.sync_copy(ones_t, deg_sh.at[dst_t], add=True)
        plsc.subcore_barrier()
        _tile_rows_copy(
            s, big, last,
            lambda: pltpu.sync_copy(deg_sh.at[pl.ds(row0, big)],
                                    out_hbm.at[c, pl.ds(row0, big)]),
            lambda: pltpu.sync_copy(deg_sh.at[pl.ds(row0, last)],
                                    out_hbm.at[c, pl.ds(row0, last)]))

    return deg_kernel


def _make_agg_kernel(E, n, F):
    EC = E // NC
    ET = EC // NS
    assert EC * NC == E and ET * NS == EC
    n_full = ET // CH
    tail = ET - n_full * CH
    big, last = _row_split(n)

    K = 4
    n_body = n_full // K
    assert n_body * K == n_full, "pick K dividing the per-tile chunk count"

    @functools.partial(
        pl.kernel,
        out_type=jax.ShapeDtypeStruct((NC, n, F), jnp.float32),
        mesh=_sc_mesh(),
        scratch_types=[
            pltpu.VMEM_SHARED((n, F), jnp.float32),
            pltpu.VMEM((K * CH,), jnp.int32),
            [pltpu.VMEM((CH,), jnp.int32) for _ in range(K)],
            [pltpu.VMEM((CH, F), jnp.float32) for _ in range(K)],
            pltpu.VMEM((max(tail, 8),), jnp.int32),
            pltpu.VMEM((max(tail, 8),), jnp.int32),
            pltpu.VMEM((max(tail, 8), F), jnp.float32),
            pltpu.SemaphoreType.DMA,
            pltpu.SemaphoreType.DMA,
            pltpu.SemaphoreType.DMA,
        ],
    )
    def agg_kernel(hs_hbm, src_hbm, dst_hbm, zeros_hbm, out_hbm,
                   agg_sh, src_all, dst_vs, rows_vs, src_t, dst_t, rows_t,
                   sem_g, sem_d, sem_s):
        c = lax.axis_index("c")
        s = lax.axis_index("s")
        base = c * EC + s * ET
        row0 = s * big
        _tile_rows_copy(
            s, big, last,
            lambda: pltpu.sync_copy(zeros_hbm.at[pl.ds(0, big)],
                                    agg_sh.at[pl.ds(row0, big)]),
            lambda: pltpu.sync_copy(zeros_hbm.at[pl.ds(0, last)],
                                    agg_sh.at[pl.ds(row0, last)]))
        plsc.subcore_barrier()

        @pl.loop(0, n_body)
        def _(i):
            off = base + i * (K * CH)
            pltpu.sync_copy(src_hbm.at[pl.ds(off, K * CH)], src_all)
            ddescs = [
                pltpu.async_copy(dst_hbm.at[pl.ds(off + b * CH, CH)],
                                 dst_vs[b], sem_d)
                for b in range(K)
            ]
            gdescs = [
                pltpu.async_copy(hs_hbm.at[src_all.at[pl.ds(b * CH, CH)]],
                                 rows_vs[b], sem_g)
                for b in range(K)
            ]
            sdescs = []
            for b in range(K):
                ddescs[b].wait()
                gdescs[b].wait()
                sdescs.append(
                    pltpu.async_copy(rows_vs[b], agg_sh.at[dst_vs[b]],
                                     sem_s, add=True))
            for d in sdescs:
                d.wait()

        if tail:
            off = base + n_full * CH
            pltpu.sync_copy(src_hbm.at[pl.ds(off, tail)], src_t)
            pltpu.sync_copy(dst_hbm.at[pl.ds(off, tail)], dst_t)
            pltpu.async_copy(hs_hbm.at[src_t], rows_t, sem_g).wait()
            pltpu.sync_copy(rows_t, agg_sh.at[dst_t], add=True)
        plsc.subcore_barrier()
        _tile_rows_copy(
            s, big, last,
            lambda: pltpu.sync_copy(agg_sh.at[pl.ds(row0, big)],
                                    out_hbm.at[c, pl.ds(row0, big)]),
            lambda: pltpu.sync_copy(agg_sh.at[pl.ds(row0, last)],
                                    out_hbm.at[c, pl.ds(row0, last)]))

    return agg_kernel


def _b0_body(x_ref, w_ref, degp_ref, dinvb_ref, hs_ref):
    deg = degp_ref[0] + degp_ref[1]
    deg0 = deg[:, 0:1] + 1.0
    dinv = lax.rsqrt(deg0)
    dinvb = jnp.broadcast_to(dinv, (dinv.shape[0], 128))
    dinvb_ref[...] = dinvb
    h = jnp.dot(x_ref[...], w_ref[...], preferred_element_type=jnp.float32)
    hs_ref[...] = h * dinvb


def _mid_body(aggp_ref, hs_ref, dinvb_ref, b_ref, w_ref, out_ref):
    t = (aggp_ref[0] + aggp_ref[1] + hs_ref[...]) * dinvb_ref[:, : hs_ref.shape[1]]
    t = t + b_ref[...]
    t = t * jax.nn.sigmoid(t)
    h = jnp.dot(t, w_ref[...], preferred_element_type=jnp.float32)
    out_ref[...] = h * dinvb_ref[:, : out_ref.shape[1]]


def _b3_body(aggp_ref, hs_ref, dinvb_ref, b_ref, out_ref):
    w = hs_ref.shape[1]
    z = (aggp_ref[0] + aggp_ref[1] + hs_ref[...]) * dinvb_ref[:, :w] + b_ref[...]
    m = jnp.max(z, axis=0, keepdims=True)
    lse = jnp.log(jnp.sum(jnp.exp(z - m), axis=0, keepdims=True)) + m
    out_ref[...] = (z - lse)[:, : out_ref.shape[1]]


def kernel(x, edge_index, W0, b0, W1, b1, W2, b2):
    n, f_in = x.shape
    E = edge_index.shape[1]
    C = W2.shape[1]
    f32 = jnp.float32
    src = edge_index[0].astype(jnp.int32)
    dst = edge_index[1].astype(jnp.int32)

    big, _ = _row_split(n)
    ones128 = jnp.ones((CH, 128), f32)
    z128 = jnp.zeros((big, 128), f32)

    degp = _make_deg_kernel(E, n)(dst, ones128, z128)

    dinvb, hs0 = pl.pallas_call(
        _b0_body,
        out_shape=[jax.ShapeDtypeStruct((n, 128), f32),
                   jax.ShapeDtypeStruct((n, 128), f32)],
    )(x, W0, degp)

    agg128 = _make_agg_kernel(E, n, 128)
    agg0 = agg128(hs0, src, dst, z128)

    hs1 = pl.pallas_call(
        _mid_body, out_shape=jax.ShapeDtypeStruct((n, 128), f32),
    )(agg0, hs0, dinvb, b0.reshape(1, 128), W1)

    agg1 = agg128(hs1, src, dst, z128)

    W2p = jnp.pad(W2, ((0, 0), (0, 128 - C)))
    b2p = jnp.pad(b2, (0, 128 - C)).reshape(1, 128)
    hs2 = pl.pallas_call(
        _mid_body, out_shape=jax.ShapeDtypeStruct((n, 128), f32),
    )(agg1, hs1, dinvb, b1.reshape(1, 128), W2p)

    agg2 = agg128(hs2, src, dst, z128)

    out = pl.pallas_call(
        _b3_body, out_shape=jax.ShapeDtypeStruct((n, C), f32),
    )(agg2, hs2, dinvb, b2p)
    return out


# speedup vs baseline: 22.2054x; 1.0719x over previous
import functools

import jax
import jax.numpy as jnp
from jax import lax
from jax.experimental import pallas as pl
from jax.experimental.pallas import tpu as pltpu
from jax.experimental.pallas import tpu_sc as plsc

NC = 2
NS = 16
CH = 64


def _sc_mesh():
    return plsc.VectorSubcoreMesh(core_axis_name="c", subcore_axis_name="s")


def _row_split(n):
    big = -(-n // NS)
    big = -(-big // 8) * 8
    last = n - (NS - 1) * big
    assert last > 0 and last % 8 == 0 and big % 8 == 0
    return big, last


def _tile_rows_copy(s, big, last, copy_big, copy_last):
    @pl.when(s < NS - 1)
    def _():
        copy_big()

    @pl.when(s == NS - 1)
    def _():
        copy_last()


def _make_deg_kernel(E, n):
    EC = E // NC
    ET = EC // NS
    assert EC * NC == E and ET * NS == EC
    n_full = ET // CH
    tail = ET - n_full * CH
    big, last = _row_split(n)

    K = 4
    n_body = n_full // K
    assert n_body * K == n_full

    @functools.partial(
        pl.kernel,
        out_type=jax.ShapeDtypeStruct((NC, n, 128), jnp.float32),
        mesh=_sc_mesh(),
        scratch_types=[
            pltpu.VMEM_SHARED((n, 128), jnp.float32),
            pltpu.VMEM((CH, 128), jnp.float32),
            [pltpu.VMEM((CH,), jnp.int32) for _ in range(K)],
            pltpu.VMEM((max(tail, 8), 128), jnp.float32),
            pltpu.VMEM((max(tail, 8),), jnp.int32),
            pltpu.SemaphoreType.DMA,
            pltpu.SemaphoreType.DMA,
        ],
    )
    def deg_kernel(dst_hbm, ones_hbm, zeros_hbm, out_hbm,
                   deg_sh, ones_v, dst_vs, ones_t, dst_t, sem_d, sem_s):
        c = lax.axis_index("c")
        s = lax.axis_index("s")
        base = c * EC + s * ET
        row0 = s * big
        _tile_rows_copy(
            s, big, last,
            lambda: pltpu.sync_copy(zeros_hbm.at[pl.ds(0, big)],
                                    deg_sh.at[pl.ds(row0, big)]),
            lambda: pltpu.sync_copy(zeros_hbm.at[pl.ds(0, last)],
                                    deg_sh.at[pl.ds(row0, last)]))
        pltpu.sync_copy(ones_hbm, ones_v)
        if tail:
            pltpu.sync_copy(ones_hbm.at[pl.ds(0, tail)], ones_t)
        plsc.subcore_barrier()

        @pl.loop(0, n_body)
        def _(i):
            off = base + i * (K * CH)
            ddescs = [
                pltpu.async_copy(dst_hbm.at[pl.ds(off + b * CH, CH)],
                                 dst_vs[b], sem_d)
                for b in range(K)
            ]
            sdescs = []
            for b in range(K):
                ddescs[b].wait()
                sdescs.append(
                    pltpu.async_copy(ones_v, deg_sh.at[dst_vs[b]],
                                     sem_s, add=True))
            for d in sdescs:
                d.wait()

        if tail:
            off = base + n_full * CH
            pltpu.sync_copy(dst_hbm.at[pl.ds(off, tail)], dst_t)
            pltpu.sync_copy(ones_t, deg_sh.at[dst_t], add=True)
        plsc.subcore_barrier()
        _tile_rows_copy(
            s, big, last,
            lambda: pltpu.sync_copy(deg_sh.at[pl.ds(row0, big)],
                                    out_hbm.at[c, pl.ds(row0, big)]),
            lambda: pltpu.sync_copy(deg_sh.at[pl.ds(row0, last)],
                                    out_hbm.at[c, pl.ds(row0, last)]))

    return deg_kernel


def _make_agg_kernel(E, n, F):
    EC = E // NC
    ET = EC // NS
    assert EC * NC == E and ET * NS == EC
    n_full = ET // CH
    tail = ET - n_full * CH
    big, last = _row_split(n)

    K = 4
    n_body = n_full // K
    assert n_body * K == n_full, "pick K dividing the per-tile chunk count"

    @functools.partial(
        pl.kernel,
        out_type=jax.ShapeDtypeStruct((NC, n, F), jnp.float32),
        mesh=_sc_mesh(),
        scratch_types=[
            pltpu.VMEM_SHARED((n, F), jnp.float32),
            pltpu.VMEM((K * CH,), jnp.int32),
            [pltpu.VMEM((CH,), jnp.int32) for _ in range(K)],
            [pltpu.VMEM((CH, F), jnp.float32) for _ in range(K)],
            pltpu.VMEM((max(tail, 8),), jnp.int32),
            pltpu.VMEM((max(tail, 8),), jnp.int32),
            pltpu.VMEM((max(tail, 8), F), jnp.float32),
            pltpu.SemaphoreType.DMA,
            pltpu.SemaphoreType.DMA,
            pltpu.SemaphoreType.DMA,
        ],
    )
    def agg_kernel(hs_hbm, src_hbm, dst_hbm, zeros_hbm, out_hbm,
                   agg_sh, src_all, dst_vs, rows_vs, src_t, dst_t, rows_t,
                   sem_g, sem_d, sem_s):
        c = lax.axis_index("c")
        s = lax.axis_index("s")
        base = c * EC + s * ET
        row0 = s * big
        _tile_rows_copy(
            s, big, last,
            lambda: pltpu.sync_copy(zeros_hbm.at[pl.ds(0, big)],
                                    agg_sh.at[pl.ds(row0, big)]),
            lambda: pltpu.sync_copy(zeros_hbm.at[pl.ds(0, last)],
                                    agg_sh.at[pl.ds(row0, last)]))
        plsc.subcore_barrier()

        def drain_scatter(b):
            pltpu.make_async_copy(hs_hbm.at[pl.ds(0, CH)], rows_vs[b],
                                  sem_s).wait()

        @pl.loop(0, n_body)
        def _(i):
            off = base + i * (K * CH)
            pltpu.sync_copy(src_hbm.at[pl.ds(off, K * CH)], src_all)
            ddescs, gdescs = [], []
            for b in range(K):
                @pl.when(i > 0)
                def _():
                    drain_scatter(b)
                ddescs.append(
                    pltpu.async_copy(dst_hbm.at[pl.ds(off + b * CH, CH)],
                                     dst_vs[b], sem_d))
                gdescs.append(
                    pltpu.async_copy(hs_hbm.at[src_all.at[pl.ds(b * CH, CH)]],
                                     rows_vs[b], sem_g))
            for b in range(K):
                ddescs[b].wait()
                gdescs[b].wait()
                pltpu.async_copy(rows_vs[b], agg_sh.at[dst_vs[b]],
                                 sem_s, add=True)

        for b in range(K):
            drain_scatter(b)

        if tail:
            off = base + n_full * CH
            pltpu.sync_copy(src_hbm.at[pl.ds(off, tail)], src_t)
            pltpu.sync_copy(dst_hbm.at[pl.ds(off, tail)], dst_t)
            pltpu.async_copy(hs_hbm.at[src_t], rows_t, sem_g).wait()
            pltpu.sync_copy(rows_t, agg_sh.at[dst_t], add=True)
        plsc.subcore_barrier()
        _tile_rows_copy(
            s, big, last,
            lambda: pltpu.sync_copy(agg_sh.at[pl.ds(row0, big)],
                                    out_hbm.at[c, pl.ds(row0, big)]),
            lambda: pltpu.sync_copy(agg_sh.at[pl.ds(row0, last)],
                                    out_hbm.at[c, pl.ds(row0, last)]))

    return agg_kernel


def _b0_body(x_ref, w_ref, degp_ref, dinvb_ref, hs_ref):
    deg = degp_ref[0] + degp_ref[1]
    deg0 = deg[:, 0:1] + 1.0
    dinv = lax.rsqrt(deg0)
    dinvb = jnp.broadcast_to(dinv, (dinv.shape[0], 128))
    dinvb_ref[...] = dinvb
    h = jnp.dot(x_ref[...], w_ref[...], preferred_element_type=jnp.float32)
    hs_ref[...] = h * dinvb


def _mid_body(aggp_ref, hs_ref, dinvb_ref, b_ref, w_ref, out_ref):
    t = (aggp_ref[0] + aggp_ref[1] + hs_ref[...]) * dinvb_ref[:, : hs_ref.shape[1]]
    t = t + b_ref[...]
    t = t * jax.nn.sigmoid(t)
    h = jnp.dot(t, w_ref[...], preferred_element_type=jnp.float32)
    out_ref[...] = h * dinvb_ref[:, : out_ref.shape[1]]


def _b3_body(aggp_ref, hs_ref, dinvb_ref, b_ref, out_ref):
    w = hs_ref.shape[1]
    z = (aggp_ref[0] + aggp_ref[1] + hs_ref[...]) * dinvb_ref[:, :w] + b_ref[...]
    m = jnp.max(z, axis=0, keepdims=True)
    lse = jnp.log(jnp.sum(jnp.exp(z - m), axis=0, keepdims=True)) + m
    out_ref[...] = (z - lse)[:, : out_ref.shape[1]]


def kernel(x, edge_index, W0, b0, W1, b1, W2, b2):
    n, f_in = x.shape
    E = edge_index.shape[1]
    C = W2.shape[1]
    f32 = jnp.float32
    src = edge_index[0].astype(jnp.int32)
    dst = edge_index[1].astype(jnp.int32)

    big, _ = _row_split(n)
    ones128 = jnp.ones((CH, 128), f32)
    z128 = jnp.zeros((big, 128), f32)

    degp = _make_deg_kernel(E, n)(dst, ones128, z128)

    dinvb, hs0 = pl.pallas_call(
        _b0_body,
        out_shape=[jax.ShapeDtypeStruct((n, 128), f32),
                   jax.ShapeDtypeStruct((n, 128), f32)],
    )(x, W0, degp)

    agg128 = _make_agg_kernel(E, n, 128)
    agg0 = agg128(hs0, src, dst, z128)

    hs1 = pl.pallas_call(
        _mid_body, out_shape=jax.ShapeDtypeStruct((n, 128), f32),
    )(agg0, hs0, dinvb, b0.reshape(1, 128), W1)

    agg1 = agg128(hs1, src, dst, z128)

    W2p = jnp.pad(W2, ((0, 0), (0, 128 - C)))
    b2p = jnp.pad(b2, (0, 128 - C)).reshape(1, 128)
    hs2 = pl.pallas_call(
        _mid_body, out_shape=jax.ShapeDtypeStruct((n, 128), f32),
    )(agg1, hs1, dinvb, b1.reshape(1, 128), W2p)

    agg2 = agg128(hs2, src, dst, z128)

    out = pl.pallas_call(
        _b3_body, out_shape=jax.ShapeDtypeStruct((n, C), f32),
    )(agg2, hs2, dinvb, b2p)
    return out


# speedup vs baseline: 22.3582x; 1.0069x over previous
import functools

import jax
import jax.numpy as jnp
from jax import lax
from jax.experimental import pallas as pl
from jax.experimental.pallas import tpu as pltpu
from jax.experimental.pallas import tpu_sc as plsc

NC = 2
NS = 16
CH = 128


def _sc_mesh():
    return plsc.VectorSubcoreMesh(core_axis_name="c", subcore_axis_name="s")


def _row_split(n):
    big = -(-n // NS)
    big = -(-big // 8) * 8
    last = n - (NS - 1) * big
    assert last > 0 and last % 8 == 0 and big % 8 == 0
    return big, last


def _tile_rows_copy(s, big, last, copy_big, copy_last):
    @pl.when(s < NS - 1)
    def _():
        copy_big()

    @pl.when(s == NS - 1)
    def _():
        copy_last()


def _make_deg_kernel(E, n):
    EC = E // NC
    ET = EC // NS
    assert EC * NC == E and ET * NS == EC
    n_full = ET // CH
    tail = ET - n_full * CH
    big, last = _row_split(n)

    K = 3
    n_body = n_full // K
    assert n_body * K == n_full

    @functools.partial(
        pl.kernel,
        out_type=jax.ShapeDtypeStruct((NC, n, 128), jnp.float32),
        mesh=_sc_mesh(),
        scratch_types=[
            pltpu.VMEM_SHARED((n, 128), jnp.float32),
            pltpu.VMEM((CH, 128), jnp.float32),
            [pltpu.VMEM((CH,), jnp.int32) for _ in range(K)],
            pltpu.VMEM((max(tail, 8), 128), jnp.float32),
            pltpu.VMEM((max(tail, 8),), jnp.int32),
            pltpu.SemaphoreType.DMA,
            pltpu.SemaphoreType.DMA,
        ],
    )
    def deg_kernel(dst_hbm, ones_hbm, zeros_hbm, out_hbm,
                   deg_sh, ones_v, dst_vs, ones_t, dst_t, sem_d, sem_s):
        c = lax.axis_index("c")
        s = lax.axis_index("s")
        base = c * EC + s * ET
        row0 = s * big
        _tile_rows_copy(
            s, big, last,
            lambda: pltpu.sync_copy(zeros_hbm.at[pl.ds(0, big)],
                                    deg_sh.at[pl.ds(row0, big)]),
            lambda: pltpu.sync_copy(zeros_hbm.at[pl.ds(0, last)],
                                    deg_sh.at[pl.ds(row0, last)]))
        pltpu.sync_copy(ones_hbm, ones_v)
        if tail:
            pltpu.sync_copy(ones_hbm.at[pl.ds(0, tail)], ones_t)
        plsc.subcore_barrier()

        @pl.loop(0, n_body)
        def _(i):
            off = base + i * (K * CH)
            ddescs = [
                pltpu.async_copy(dst_hbm.at[pl.ds(off + b * CH, CH)],
                                 dst_vs[b], sem_d)
                for b in range(K)
            ]
            sdescs = []
            for b in range(K):
                ddescs[b].wait()
                sdescs.append(
                    pltpu.async_copy(ones_v, deg_sh.at[dst_vs[b]],
                                     sem_s, add=True))
            for d in sdescs:
                d.wait()

        if tail:
            off = base + n_full * CH
            pltpu.sync_copy(dst_hbm.at[pl.ds(off, tail)], dst_t)
            pltpu.sync_copy(ones_t, deg_sh.at[dst_t], add=True)
        plsc.subcore_barrier()
        _tile_rows_copy(
            s, big, last,
            lambda: pltpu.sync_copy(deg_sh.at[pl.ds(row0, big)],
                                    out_hbm.at[c, pl.ds(row0, big)]),
            lambda: pltpu.sync_copy(deg_sh.at[pl.ds(row0, last)],
                                    out_hbm.at[c, pl.ds(row0, last)]))

    return deg_kernel


def _make_agg_kernel(E, n, F):
    EC = E // NC
    ET = EC // NS
    assert EC * NC == E and ET * NS == EC
    n_full = ET // CH
    tail = ET - n_full * CH
    big, last = _row_split(n)

    K = 2
    n_body = n_full // K
    assert n_body * K == n_full, "pick K dividing the per-tile chunk count"

    @functools.partial(
        pl.kernel,
        out_type=jax.ShapeDtypeStruct((NC, n, F), jnp.float32),
        mesh=_sc_mesh(),
        scratch_types=[
            pltpu.VMEM_SHARED((n, F), jnp.float32),
            pltpu.VMEM((K * CH,), jnp.int32),
            [pltpu.VMEM((CH,), jnp.int32) for _ in range(K)],
            [pltpu.VMEM((CH, F), jnp.float32) for _ in range(K)],
            pltpu.VMEM((max(tail, 8),), jnp.int32),
            pltpu.VMEM((max(tail, 8),), jnp.int32),
            pltpu.VMEM((max(tail, 8), F), jnp.float32),
            pltpu.SemaphoreType.DMA,
            pltpu.SemaphoreType.DMA,
            pltpu.SemaphoreType.DMA,
        ],
    )
    def agg_kernel(hs_hbm, src_hbm, dst_hbm, zeros_hbm, out_hbm,
                   agg_sh, src_all, dst_vs, rows_vs, src_t, dst_t, rows_t,
                   sem_g, sem_d, sem_s):
        c = lax.axis_index("c")
        s = lax.axis_index("s")
        base = c * EC + s * ET
        row0 = s * big
        _tile_rows_copy(
            s, big, last,
            lambda: pltpu.sync_copy(zeros_hbm.at[pl.ds(0, big)],
                                    agg_sh.at[pl.ds(row0, big)]),
            lambda: pltpu.sync_copy(zeros_hbm.at[pl.ds(0, last)],
                                    agg_sh.at[pl.ds(row0, last)]))
        plsc.subcore_barrier()

        def drain_scatter(b):
            pltpu.make_async_copy(hs_hbm.at[pl.ds(0, CH)], rows_vs[b],
                                  sem_s).wait()

        @pl.loop(0, n_body)
        def _(i):
            off = base + i * (K * CH)
            pltpu.sync_copy(src_hbm.at[pl.ds(off, K * CH)], src_all)
            ddescs, gdescs = [], []
            for b in range(K):
                @pl.when(i > 0)
                def _():
                    drain_scatter(b)
                ddescs.append(
                    pltpu.async_copy(dst_hbm.at[pl.ds(off + b * CH, CH)],
                                     dst_vs[b], sem_d))
                gdescs.append(
                    pltpu.async_copy(hs_hbm.at[src_all.at[pl.ds(b * CH, CH)]],
                                     rows_vs[b], sem_g))
            for b in range(K):
                ddescs[b].wait()
                gdescs[b].wait()
                pltpu.async_copy(rows_vs[b], agg_sh.at[dst_vs[b]],
                                 sem_s, add=True)

        for b in range(K):
            drain_scatter(b)

        if tail:
            off = base + n_full * CH
            pltpu.sync_copy(src_hbm.at[pl.ds(off, tail)], src_t)
            pltpu.sync_copy(dst_hbm.at[pl.ds(off, tail)], dst_t)
            pltpu.async_copy(hs_hbm.at[src_t], rows_t, sem_g).wait()
            pltpu.sync_copy(rows_t, agg_sh.at[dst_t], add=True)
        plsc.subcore_barrier()
        _tile_rows_copy(
            s, big, last,
            lambda: pltpu.sync_copy(agg_sh.at[pl.ds(row0, big)],
                                    out_hbm.at[c, pl.ds(row0, big)]),
            lambda: pltpu.sync_copy(agg_sh.at[pl.ds(row0, last)],
                                    out_hbm.at[c, pl.ds(row0, last)]))

    return agg_kernel


def _b0_body(x_ref, w_ref, degp_ref, dinvb_ref, hs_ref):
    deg = degp_ref[0] + degp_ref[1]
    deg0 = deg[:, 0:1] + 1.0
    dinv = lax.rsqrt(deg0)
    dinvb = jnp.broadcast_to(dinv, (dinv.shape[0], 128))
    dinvb_ref[...] = dinvb
    h = jnp.dot(x_ref[...], w_ref[...], preferred_element_type=jnp.float32)
    hs_ref[...] = h * dinvb


def _mid_body(aggp_ref, hs_ref, dinvb_ref, b_ref, w_ref, out_ref):
    t = (aggp_ref[0] + aggp_ref[1] + hs_ref[...]) * dinvb_ref[:, : hs_ref.shape[1]]
    t = t + b_ref[...]
    t = t * jax.nn.sigmoid(t)
    h = jnp.dot(t, w_ref[...], preferred_element_type=jnp.float32)
    out_ref[...] = h * dinvb_ref[:, : out_ref.shape[1]]


def _b3_body(aggp_ref, hs_ref, dinvb_ref, b_ref, out_ref):
    w = hs_ref.shape[1]
    z = (aggp_ref[0] + aggp_ref[1] + hs_ref[...]) * dinvb_ref[:, :w] + b_ref[...]
    m = jnp.max(z, axis=0, keepdims=True)
    lse = jnp.log(jnp.sum(jnp.exp(z - m), axis=0, keepdims=True)) + m
    out_ref[...] = (z - lse)[:, : out_ref.shape[1]]


def kernel(x, edge_index, W0, b0, W1, b1, W2, b2):
    n, f_in = x.shape
    E = edge_index.shape[1]
    C = W2.shape[1]
    f32 = jnp.float32
    src = edge_index[0].astype(jnp.int32)
    dst = edge_index[1].astype(jnp.int32)

    big, _ = _row_split(n)
    ones128 = jnp.ones((CH, 128), f32)
    z128 = jnp.zeros((big, 128), f32)

    degp = _make_deg_kernel(E, n)(dst, ones128, z128)

    dinvb, hs0 = pl.pallas_call(
        _b0_body,
        out_shape=[jax.ShapeDtypeStruct((n, 128), f32),
                   jax.ShapeDtypeStruct((n, 128), f32)],
    )(x, W0, degp)

    agg128 = _make_agg_kernel(E, n, 128)
    agg0 = agg128(hs0, src, dst, z128)

    hs1 = pl.pallas_call(
        _mid_body, out_shape=jax.ShapeDtypeStruct((n, 128), f32),
    )(agg0, hs0, dinvb, b0.reshape(1, 128), W1)

    agg1 = agg128(hs1, src, dst, z128)

    W2p = jnp.pad(W2, ((0, 0), (0, 128 - C)))
    b2p = jnp.pad(b2, (0, 128 - C)).reshape(1, 128)
    hs2 = pl.pallas_call(
        _mid_body, out_shape=jax.ShapeDtypeStruct((n, 128), f32),
    )(agg1, hs1, dinvb, b1.reshape(1, 128), W2p)

    agg2 = agg128(hs2, src, dst, z128)

    out = pl.pallas_call(
        _b3_body, out_shape=jax.ShapeDtypeStruct((n, C), f32),
    )(agg2, hs2, dinvb, b2p)
    return out


# speedup vs baseline: 24.2731x; 1.0857x over previous
import functools

import jax
import jax.numpy as jnp
from jax import lax
from jax.experimental import pallas as pl
from jax.experimental.pallas import tpu as pltpu
from jax.experimental.pallas import tpu_sc as plsc

NC = 2
NS = 16
CH = 128


def _sc_mesh():
    return plsc.VectorSubcoreMesh(core_axis_name="c", subcore_axis_name="s")


def _row_split(n):
    big = -(-n // NS)
    big = -(-big // 8) * 8
    last = n - (NS - 1) * big
    assert last > 0 and last % 8 == 0 and big % 8 == 0
    return big, last


def _tile_rows_copy(s, big, last, copy_big, copy_last):
    @pl.when(s < NS - 1)
    def _():
        copy_big()

    @pl.when(s == NS - 1)
    def _():
        copy_last()


def _make_deg_kernel(E, n):
    EC = E // NC
    ET = EC // NS
    assert EC * NC == E and ET * NS == EC
    n_full = ET // CH
    tail = ET - n_full * CH
    big, last = _row_split(n)

    K = 3
    n_body = n_full // K
    assert n_body * K == n_full

    @functools.partial(
        pl.kernel,
        out_type=jax.ShapeDtypeStruct((NC, n, 128), jnp.float32),
        mesh=_sc_mesh(),
        scratch_types=[
            pltpu.VMEM_SHARED((n, 128), jnp.float32),
            pltpu.VMEM((CH, 128), jnp.float32),
            [pltpu.VMEM((CH,), jnp.int32) for _ in range(K)],
            pltpu.VMEM((max(tail, 8), 128), jnp.float32),
            pltpu.VMEM((max(tail, 8),), jnp.int32),
            pltpu.SemaphoreType.DMA,
            pltpu.SemaphoreType.DMA,
        ],
    )
    def deg_kernel(dst_hbm, ones_hbm, zeros_hbm, out_hbm,
                   deg_sh, ones_v, dst_vs, ones_t, dst_t, sem_d, sem_s):
        c = lax.axis_index("c")
        s = lax.axis_index("s")
        base = c * EC + s * ET
        row0 = s * big
        _tile_rows_copy(
            s, big, last,
            lambda: pltpu.sync_copy(zeros_hbm.at[pl.ds(0, big)],
                                    deg_sh.at[pl.ds(row0, big)]),
            lambda: pltpu.sync_copy(zeros_hbm.at[pl.ds(0, last)],
                                    deg_sh.at[pl.ds(row0, last)]))
        pltpu.sync_copy(ones_hbm, ones_v)
        if tail:
            pltpu.sync_copy(ones_hbm.at[pl.ds(0, tail)], ones_t)
        plsc.subcore_barrier()

        @pl.loop(0, n_body)
        def _(i):
            off = base + i * (K * CH)
            ddescs = [
                pltpu.async_copy(dst_hbm.at[pl.ds(off + b * CH, CH)],
                                 dst_vs[b], sem_d)
                for b in range(K)
            ]
            sdescs = []
            for b in range(K):
                ddescs[b].wait()
                sdescs.append(
                    pltpu.async_copy(ones_v, deg_sh.at[dst_vs[b]],
                                     sem_s, add=True))
            for d in sdescs:
                d.wait()

        if tail:
            off = base + n_full * CH
            pltpu.sync_copy(dst_hbm.at[pl.ds(off, tail)], dst_t)
            pltpu.sync_copy(ones_t, deg_sh.at[dst_t], add=True)
        plsc.subcore_barrier()
        _tile_rows_copy(
            s, big, last,
            lambda: pltpu.sync_copy(deg_sh.at[pl.ds(row0, big)],
                                    out_hbm.at[c, pl.ds(row0, big)]),
            lambda: pltpu.sync_copy(deg_sh.at[pl.ds(row0, last)],
                                    out_hbm.at[c, pl.ds(row0, last)]))

    return deg_kernel


def _make_agg_kernel(E, n, F):
    EC = E // NC
    ET = EC // NS
    assert EC * NC == E and ET * NS == EC
    n_full = ET // CH
    tail = ET - n_full * CH
    big, last = _row_split(n)

    K = 2
    n_body = n_full // K
    assert n_body * K == n_full, "pick K dividing the per-tile chunk count"

    @functools.partial(
        pl.kernel,
        out_type=jax.ShapeDtypeStruct((NC, n, F), jnp.float32),
        mesh=_sc_mesh(),
        scratch_types=[
            pltpu.VMEM_SHARED((n, F), jnp.float32),
            pltpu.VMEM((ET,), jnp.int32),
            [pltpu.VMEM((CH,), jnp.int32) for _ in range(K)],
            [pltpu.VMEM((CH, F), jnp.float32) for _ in range(K)],
            pltpu.VMEM((max(tail, 8),), jnp.int32),
            pltpu.VMEM((max(tail, 8), F), jnp.float32),
            pltpu.SemaphoreType.DMA,
            pltpu.SemaphoreType.DMA,
            pltpu.SemaphoreType.DMA,
        ],
    )
    def agg_kernel(hs_hbm, src_hbm, dst_hbm, zeros_hbm, out_hbm,
                   agg_sh, src_all, dst_vs, rows_vs, dst_t, rows_t,
                   sem_g, sem_d, sem_s):
        c = lax.axis_index("c")
        s = lax.axis_index("s")
        base = c * EC + s * ET
        row0 = s * big
        src_desc = pltpu.async_copy(src_hbm.at[pl.ds(base, ET)], src_all,
                                    sem_d)
        _tile_rows_copy(
            s, big, last,
            lambda: pltpu.sync_copy(zeros_hbm.at[pl.ds(0, big)],
                                    agg_sh.at[pl.ds(row0, big)]),
            lambda: pltpu.sync_copy(zeros_hbm.at[pl.ds(0, last)],
                                    agg_sh.at[pl.ds(row0, last)]))
        src_desc.wait()
        plsc.subcore_barrier()

        def drain_scatter(b):
            pltpu.make_async_copy(hs_hbm.at[pl.ds(0, CH)], rows_vs[b],
                                  sem_s).wait()

        @pl.loop(0, n_body)
        def _(i):
            off = base + i * (K * CH)
            loc = i * (K * CH)
            ddescs, gdescs = [], []
            for b in range(K):
                @pl.when(i > 0)
                def _():
                    drain_scatter(b)
                ddescs.append(
                    pltpu.async_copy(dst_hbm.at[pl.ds(off + b * CH, CH)],
                                     dst_vs[b], sem_d))
                gdescs.append(
                    pltpu.async_copy(
                        hs_hbm.at[src_all.at[pl.ds(loc + b * CH, CH)]],
                        rows_vs[b], sem_g))
            for b in range(K):
                ddescs[b].wait()
                gdescs[b].wait()
                pltpu.async_copy(rows_vs[b], agg_sh.at[dst_vs[b]],
                                 sem_s, add=True)

        for b in range(K):
            drain_scatter(b)

        if tail:
            off = base + n_full * CH
            pltpu.sync_copy(dst_hbm.at[pl.ds(off, tail)], dst_t)
            pltpu.async_copy(
                hs_hbm.at[src_all.at[pl.ds(n_full * CH, tail)]],
                rows_t, sem_g).wait()
            pltpu.sync_copy(rows_t, agg_sh.at[dst_t], add=True)
        plsc.subcore_barrier()
        _tile_rows_copy(
            s, big, last,
            lambda: pltpu.sync_copy(agg_sh.at[pl.ds(row0, big)],
                                    out_hbm.at[c, pl.ds(row0, big)]),
            lambda: pltpu.sync_copy(agg_sh.at[pl.ds(row0, last)],
                                    out_hbm.at[c, pl.ds(row0, last)]))

    return agg_kernel


def _b0_body(x_ref, w_ref, degp_ref, dinvb_ref, hs_ref):
    deg = degp_ref[0] + degp_ref[1]
    deg0 = deg[:, 0:1] + 1.0
    dinv = lax.rsqrt(deg0)
    dinvb = jnp.broadcast_to(dinv, (dinv.shape[0], 128))
    dinvb_ref[...] = dinvb
    h = jnp.dot(x_ref[...], w_ref[...], preferred_element_type=jnp.float32)
    hs_ref[...] = h * dinvb


def _mid_body(aggp_ref, hs_ref, dinvb_ref, b_ref, w_ref, out_ref):
    t = (aggp_ref[0] + aggp_ref[1] + hs_ref[...]) * dinvb_ref[:, : hs_ref.shape[1]]
    t = t + b_ref[...]
    t = t * jax.nn.sigmoid(t)
    h = jnp.dot(t, w_ref[...], preferred_element_type=jnp.float32)
    out_ref[...] = h * dinvb_ref[:, : out_ref.shape[1]]


def _b3_body(aggp_ref, hs_ref, dinvb_ref, b_ref, out_ref):
    w = hs_ref.shape[1]
    z = (aggp_ref[0] + aggp_ref[1] + hs_ref[...]) * dinvb_ref[:, :w] + b_ref[...]
    m = jnp.max(z, axis=0, keepdims=True)
    lse = jnp.log(jnp.sum(jnp.exp(z - m), axis=0, keepdims=True)) + m
    out_ref[...] = (z - lse)[:, : out_ref.shape[1]]


def kernel(x, edge_index, W0, b0, W1, b1, W2, b2):
    n, f_in = x.shape
    E = edge_index.shape[1]
    C = W2.shape[1]
    f32 = jnp.float32
    src = edge_index[0].astype(jnp.int32)
    dst = edge_index[1].astype(jnp.int32)

    big, _ = _row_split(n)
    ones128 = jnp.ones((CH, 128), f32)
    z128 = jnp.zeros((big, 128), f32)

    degp = _make_deg_kernel(E, n)(dst, ones128, z128)

    dinvb, hs0 = pl.pallas_call(
        _b0_body,
        out_shape=[jax.ShapeDtypeStruct((n, 128), f32),
                   jax.ShapeDtypeStruct((n, 128), f32)],
    )(x, W0, degp)

    agg128 = _make_agg_kernel(E, n, 128)
    agg0 = agg128(hs0, src, dst, z128)

    hs1 = pl.pallas_call(
        _mid_body, out_shape=jax.ShapeDtypeStruct((n, 128), f32),
    )(agg0, hs0, dinvb, b0.reshape(1, 128), W1)

    agg1 = agg128(hs1, src, dst, z128)

    W2p = jnp.pad(W2, ((0, 0), (0, 128 - C)))
    b2p = jnp.pad(b2, (0, 128 - C)).reshape(1, 128)
    hs2 = pl.pallas_call(
        _mid_body, out_shape=jax.ShapeDtypeStruct((n, 128), f32),
    )(agg1, hs1, dinvb, b1.reshape(1, 128), W2p)

    agg2 = agg128(hs2, src, dst, z128)

    out = pl.pallas_call(
        _b3_body, out_shape=jax.ShapeDtypeStruct((n, C), f32),
    )(agg2, hs2, dinvb, b2p)
    return out


# speedup vs baseline: 27.5490x; 1.1350x over previous
import functools

import jax
import jax.numpy as jnp
from jax import lax
from jax.experimental import pallas as pl
from jax.experimental.pallas import tpu as pltpu
from jax.experimental.pallas import tpu_sc as plsc

NC = 2
NS = 16
CH = 128


def _sc_mesh():
    return plsc.VectorSubcoreMesh(core_axis_name="c", subcore_axis_name="s")


def _row_split(n):
    big = -(-n // NS)
    big = -(-big // 8) * 8
    last = n - (NS - 1) * big
    assert last > 0 and last % 8 == 0 and big % 8 == 0
    return big, last


def _tile_rows_copy(s, big, last, copy_big, copy_last):
    @pl.when(s < NS - 1)
    def _():
        copy_big()

    @pl.when(s == NS - 1)
    def _():
        copy_last()


def _make_deg_kernel(E, n):
    EC = E // NC
    ET = EC // NS
    assert EC * NC == E and ET * NS == EC
    n_full = ET // CH
    tail = ET - n_full * CH

    K = 3
    n_body = n_full // K
    assert n_body * K == n_full

    @functools.partial(
        pl.kernel,
        out_type=jax.ShapeDtypeStruct((NC, n), jnp.float32),
        mesh=_sc_mesh(),
        scratch_types=[
            pltpu.VMEM_SHARED((n,), jnp.float32),
            pltpu.VMEM((CH,), jnp.float32),
            [pltpu.VMEM((CH,), jnp.int32) for _ in range(K)],
            pltpu.VMEM((max(tail, 8),), jnp.float32),
            pltpu.VMEM((max(tail, 8),), jnp.int32),
            pltpu.VMEM((n,), jnp.float32),
            pltpu.SemaphoreType.DMA,
            pltpu.SemaphoreType.DMA,
        ],
    )
    def deg_kernel(dst_hbm, ones_hbm, zeros_hbm, out_hbm,
                   deg_sh, ones_v, dst_vs, ones_t, dst_t, zed_v,
                   sem_d, sem_s):
        c = lax.axis_index("c")
        s = lax.axis_index("s")
        base = c * EC + s * ET

        @pl.when(s == 0)
        def _():
            pltpu.sync_copy(zeros_hbm, zed_v)
            pltpu.sync_copy(zed_v, deg_sh)
        pltpu.sync_copy(ones_hbm, ones_v)
        if tail:
            pltpu.sync_copy(ones_hbm.at[pl.ds(0, tail)], ones_t)
        plsc.subcore_barrier()

        @pl.loop(0, n_body)
        def _(i):
            off = base + i * (K * CH)
            ddescs = [
                pltpu.async_copy(dst_hbm.at[pl.ds(off + b * CH, CH)],
                                 dst_vs[b], sem_d)
                for b in range(K)
            ]
            sdescs = []
            for b in range(K):
                ddescs[b].wait()
                sdescs.append(
                    pltpu.async_copy(ones_v, deg_sh.at[dst_vs[b]],
                                     sem_s, add=True))
            for d in sdescs:
                d.wait()

        if tail:
            off = base + n_full * CH
            pltpu.sync_copy(dst_hbm.at[pl.ds(off, tail)], dst_t)
            pltpu.sync_copy(ones_t, deg_sh.at[dst_t], add=True)
        plsc.subcore_barrier()

        @pl.when(s == 0)
        def _():
            pltpu.sync_copy(deg_sh, out_hbm.at[c])

    return deg_kernel


def _make_agg_kernel(E, n, F):
    EC = E // NC
    ET = EC // NS
    assert EC * NC == E and ET * NS == EC
    n_full = ET // CH
    tail = ET - n_full * CH
    big, last = _row_split(n)

    K = 2
    n_body = n_full // K
    assert n_body * K == n_full, "pick K dividing the per-tile chunk count"

    @functools.partial(
        pl.kernel,
        out_type=jax.ShapeDtypeStruct((NC, n, F), jnp.float32),
        mesh=_sc_mesh(),
        scratch_types=[
            pltpu.VMEM_SHARED((n, F), jnp.float32),
            pltpu.VMEM((ET,), jnp.int32),
            [pltpu.VMEM((CH,), jnp.int32) for _ in range(K)],
            [pltpu.VMEM((CH, F), jnp.float32) for _ in range(K)],
            pltpu.VMEM((max(tail, 8),), jnp.int32),
            pltpu.VMEM((max(tail, 8), F), jnp.float32),
            pltpu.SemaphoreType.DMA,
            pltpu.SemaphoreType.DMA,
            pltpu.SemaphoreType.DMA,
        ],
    )
    def agg_kernel(hs_hbm, src_hbm, dst_hbm, zeros_hbm, out_hbm,
                   agg_sh, src_all, dst_vs, rows_vs, dst_t, rows_t,
                   sem_g, sem_d, sem_s):
        c = lax.axis_index("c")
        s = lax.axis_index("s")
        base = c * EC + s * ET
        row0 = s * big
        src_desc = pltpu.async_copy(src_hbm.at[pl.ds(base, ET)], src_all,
                                    sem_d)
        _tile_rows_copy(
            s, big, last,
            lambda: pltpu.sync_copy(zeros_hbm.at[pl.ds(0, big)],
                                    agg_sh.at[pl.ds(row0, big)]),
            lambda: pltpu.sync_copy(zeros_hbm.at[pl.ds(0, last)],
                                    agg_sh.at[pl.ds(row0, last)]))
        src_desc.wait()
        plsc.subcore_barrier()

        def drain_scatter(b):
            pltpu.make_async_copy(hs_hbm.at[pl.ds(0, CH)], rows_vs[b],
                                  sem_s).wait()

        @pl.loop(0, n_body)
        def _(i):
            off = base + i * (K * CH)
            loc = i * (K * CH)
            ddescs, gdescs = [], []
            for b in range(K):
                @pl.when(i > 0)
                def _():
                    drain_scatter(b)
                ddescs.append(
                    pltpu.async_copy(dst_hbm.at[pl.ds(off + b * CH, CH)],
                                     dst_vs[b], sem_d))
                gdescs.append(
                    pltpu.async_copy(
                        hs_hbm.at[src_all.at[pl.ds(loc + b * CH, CH)]],
                        rows_vs[b], sem_g))
            for b in range(K):
                ddescs[b].wait()
                gdescs[b].wait()
                pltpu.async_copy(rows_vs[b], agg_sh.at[dst_vs[b]],
                                 sem_s, add=True)

        for b in range(K):
            drain_scatter(b)

        if tail:
            off = base + n_full * CH
            pltpu.sync_copy(dst_hbm.at[pl.ds(off, tail)], dst_t)
            pltpu.async_copy(
                hs_hbm.at[src_all.at[pl.ds(n_full * CH, tail)]],
                rows_t, sem_g).wait()
            pltpu.sync_copy(rows_t, agg_sh.at[dst_t], add=True)
        plsc.subcore_barrier()
        _tile_rows_copy(
            s, big, last,
            lambda: pltpu.sync_copy(agg_sh.at[pl.ds(row0, big)],
                                    out_hbm.at[c, pl.ds(row0, big)]),
            lambda: pltpu.sync_copy(agg_sh.at[pl.ds(row0, last)],
                                    out_hbm.at[c, pl.ds(row0, last)]))

    return agg_kernel


def _b0_body(x_ref, w_ref, degp_ref, dinvb_ref, hs_ref):
    n = degp_ref.shape[1]
    degp = degp_ref[...]
    deg_row = degp[0:1, :] + degp[1:2, :] + 1.0
    dinv = jnp.transpose(lax.rsqrt(deg_row), (1, 0))
    dinvb = jnp.broadcast_to(dinv, (n, 128))
    dinvb_ref[...] = dinvb
    h = jnp.dot(x_ref[...], w_ref[...], preferred_element_type=jnp.float32)
    hs_ref[...] = h * dinvb


def _mid_body(aggp_ref, hs_ref, dinvb_ref, b_ref, w_ref, out_ref):
    t = (aggp_ref[0] + aggp_ref[1] + hs_ref[...]) * dinvb_ref[:, : hs_ref.shape[1]]
    t = t + b_ref[...]
    t = t * jax.nn.sigmoid(t)
    h = jnp.dot(t, w_ref[...], preferred_element_type=jnp.float32)
    out_ref[...] = h * dinvb_ref[:, : out_ref.shape[1]]


def _b3_body(aggp_ref, hs_ref, dinvb_ref, b_ref, out_ref):
    w = hs_ref.shape[1]
    z = (aggp_ref[0] + aggp_ref[1] + hs_ref[...]) * dinvb_ref[:, :w] + b_ref[...]
    m = jnp.max(z, axis=0, keepdims=True)
    lse = jnp.log(jnp.sum(jnp.exp(z - m), axis=0, keepdims=True)) + m
    out_ref[...] = (z - lse)[:, : out_ref.shape[1]]


def kernel(x, edge_index, W0, b0, W1, b1, W2, b2):
    n, f_in = x.shape
    E = edge_index.shape[1]
    C = W2.shape[1]
    f32 = jnp.float32
    src = edge_index[0].astype(jnp.int32)
    dst = edge_index[1].astype(jnp.int32)

    big, _ = _row_split(n)
    ones_ch = jnp.ones((CH,), f32)
    z128 = jnp.zeros((big, 128), f32)
    zn = jnp.zeros((n,), f32)

    degp = _make_deg_kernel(E, n)(dst, ones_ch, zn)

    dinvb, hs0 = pl.pallas_call(
        _b0_body,
        out_shape=[jax.ShapeDtypeStruct((n, 128), f32),
                   jax.ShapeDtypeStruct((n, 128), f32)],
    )(x, W0, degp)

    agg128 = _make_agg_kernel(E, n, 128)
    agg0 = agg128(hs0, src, dst, z128)

    hs1 = pl.pallas_call(
        _mid_body, out_shape=jax.ShapeDtypeStruct((n, 128), f32),
    )(agg0, hs0, dinvb, b0.reshape(1, 128), W1)

    agg1 = agg128(hs1, src, dst, z128)

    W2p = jnp.pad(W2, ((0, 0), (0, 128 - C)))
    b2p = jnp.pad(b2, (0, 128 - C)).reshape(1, 128)
    hs2 = pl.pallas_call(
        _mid_body, out_shape=jax.ShapeDtypeStruct((n, 128), f32),
    )(agg1, hs1, dinvb, b1.reshape(1, 128), W2p)

    agg2 = agg128(hs2, src, dst, z128)

    out = pl.pallas_call(
        _b3_body, out_shape=jax.ShapeDtypeStruct((n, C), f32),
    )(agg2, hs2, dinvb, b2p)
    return out
